# baseline (speedup 1.0000x reference)
import functools

import jax
import jax.numpy as jnp
from jax import lax
from jax.experimental import pallas as pl
from jax.experimental.pallas import tpu as pltpu
from jax.experimental.pallas import tpu_sc as plsc

HIDDEN = 128
VOCAB = 65539
EMB = 51

_NC, _NS, _L = 2, 16, 16
_NW = _NC * _NS
_CH = 128
_G = HIDDEN // _L


def _proj_body(e0, e1, e2, w0, w1, w2, o0, o1, o2):
    dn = (((1,), (0,)), ((), ()))
    o0[...] = lax.dot_general(e0[...], w0[...], dn,
                              preferred_element_type=jnp.float32)
    o1[...] = lax.dot_general(e1[...], w1[...], dn,
                              preferred_element_type=jnp.float32)
    o2[...] = lax.dot_general(e2[...], w2[...], dn,
                              preferred_element_type=jnp.float32)


def _project_tables(emb0, emb1, emb2, w0, w1, w2):
    R = 4096
    nblk = (VOCAB + R - 1) // R
    espec = pl.BlockSpec((R, EMB), lambda i: (i, 0))
    wspec = pl.BlockSpec((EMB, HIDDEN), lambda i: (0, 0))
    ospec = pl.BlockSpec((R, HIDDEN), lambda i: (i, 0))
    oshape = jax.ShapeDtypeStruct((VOCAB, HIDDEN), jnp.float32)
    return pl.pallas_call(
        _proj_body,
        grid=(nblk,),
        in_specs=[espec, espec, espec, wspec, wspec, wspec],
        out_specs=[ospec, ospec, ospec],
        out_shape=[oshape, oshape, oshape],
    )(emb0, emb1, emb2, w0, w1, w2)


def _split_body(x_ref, s0, s1, s2, o0, o1, o2):
    dn = (((1,), (0,)), ((), ()))
    hi = lax.Precision.HIGHEST
    x = x_ref[...]
    half = jnp.float32(0.5)
    o0[...] = (lax.dot_general(x, s0[...], dn, precision=hi,
                               preferred_element_type=jnp.float32)
               + half).astype(jnp.int32)
    o1[...] = (lax.dot_general(x, s1[...], dn, precision=hi,
                               preferred_element_type=jnp.float32)
               + half).astype(jnp.int32)
    o2[...] = (lax.dot_general(x, s2[...], dn, precision=hi,
                               preferred_element_type=jnp.float32)
               + half).astype(jnp.int32)


def _split_input(inw, s0, s1, s2, ntok):
    RB = 512
    nrow = ntok // 128
    ishape = jax.ShapeDtypeStruct((nrow, 128), jnp.int32)
    ospec = pl.BlockSpec((RB, 128), lambda i: (i, 0))
    sspec = pl.BlockSpec((640, 128), lambda i: (0, 0))
    outs = pl.pallas_call(
        _split_body,
        grid=(nrow // RB,),
        in_specs=[pl.BlockSpec((RB, 640), lambda i: (i, 0)),
                  sspec, sspec, sspec],
        out_specs=[ospec] * 3,
        out_shape=[ishape, ishape, ishape],
    )(inw, s0, s1, s2)
    return [o.reshape(ntok) for o in outs]


def _epilogue_body(g_ref, x_ref, vtp_ref, base_ref, o_ref):
    g = g_ref[...]
    x = x_ref[...]
    rt = g.shape[0]
    acc = g + x[:, 0:1] * vtp_ref[0:1, :] + x[:, 4:5] * vtp_ref[1:2, :]
    acc = (acc.reshape(rt // 32, 32, HIDDEN) + base_ref[...][None, :, :])
    o_ref[...] = acc.reshape(rt, HIDDEN)


def _epilogue(g, in2d, vtp, base, ntok):
    RT = 4096
    return pl.pallas_call(
        _epilogue_body,
        grid=(ntok // RT,),
        in_specs=[
            pl.BlockSpec((RT, HIDDEN), lambda i: (i, 0)),
            pl.BlockSpec((RT, 5), lambda i: (i, 0)),
            pl.BlockSpec((2, HIDDEN), lambda i: (0, 0)),
            pl.BlockSpec((32, HIDDEN), lambda i: (0, 0)),
        ],
        out_specs=pl.BlockSpec((RT, HIDDEN), lambda i: (i, 0)),
        out_shape=jax.ShapeDtypeStruct((ntok, HIDDEN), jnp.float32),
    )(g, in2d, vtp, base)


def _sc_body(ntok, i0_hbm, i1_hbm, i2_hbm, t0_hbm, t1_hbm, t2_hbm, out_hbm,
             ia0, ib0, ia1, ib1, ia2, ib2,
             ra0, rb0, ra1, rb1, ra2, rb2,
             gsa, gsb, osa, osb, isa, isb):
    cid = lax.axis_index("c")
    sid = lax.axis_index("s")
    wid = sid * _NC + cid
    tpw = ntok // _NW
    nchunk = tpw // _CH
    tok0 = wid * tpw

    ibufs = ((ia0, ia1, ia2), (ib0, ib1, ib2))
    rbufs = ((ra0, ra1, ra2), (rb0, rb1, rb2))
    tabs = (t0_hbm, t1_hbm, t2_hbm)
    gsems = (gsa, gsb)
    osems = (osa, osb)
    ihbms = (i0_hbm, i1_hbm, i2_hbm)
    isems = (isa, isb)

    def start_idx(j, p):
        sl = pl.ds(tok0 + j * _CH, _CH)
        for k in range(3):
            pltpu.async_copy(ihbms[k].at[sl], ibufs[p][k], isems[p])

    def wait_idx(j, p):
        sl = pl.ds(tok0 + j * _CH, _CH)
        for k in range(3):
            pltpu.make_async_copy(ihbms[k].at[sl], ibufs[p][k],
                                  isems[p]).wait()

    def start_gathers(p):
        for k in range(3):
            pltpu.async_copy(tabs[k].at[ibufs[p][k]], rbufs[p][k], gsems[p])

    def wait_gathers(p):
        for k in range(3):
            pltpu.make_async_copy(tabs[k].at[ibufs[p][k]], rbufs[p][k],
                                  gsems[p]).wait()

    def start_out(j, p):
        pltpu.async_copy(rbufs[p][0], out_hbm.at[pl.ds(tok0 + j * _CH, _CH)],
                         osems[p])

    def wait_out(j, p):
        pltpu.make_async_copy(rbufs[p][0],
                              out_hbm.at[pl.ds(tok0 + j * _CH, _CH)],
                              osems[p]).wait()

    start_idx(0, 0)
    start_idx(1, 1)
    wait_idx(0, 0)
    start_gathers(0)

    npair = nchunk // 2

    def pair_body(jp, carry):
        for p in range(2):
            j = jp * 2 + p
            q = 1 - p

            wait_gathers(p)

            @pl.when(j + 2 < nchunk)
            def _():
                start_idx(j + 2, p)

            @pl.when(j + 1 < nchunk)
            def _():
                wait_idx(j + 1, q)

                @pl.when(j >= 1)
                def _():
                    wait_out(j - 1, q)
                start_gathers(q)

            def tok_body(r, carry2):
                for g in range(_G):
                    ds = pl.ds(g * _L, _L)
                    rbufs[p][0][r, ds] = (rbufs[p][0][r, ds]
                                          + rbufs[p][1][r, ds]
                                          + rbufs[p][2][r, ds])
                return carry2

            lax.fori_loop(0, _CH, tok_body, 0)
            start_out(j, p)
        return carry

    lax.fori_loop(0, npair, pair_body, 0)
    wait_out(nchunk - 2, 0)
    wait_out(nchunk - 1, 1)


def _sc_gather_sum(i0, i1, i2, t0, t1, t2, ntok):
    mesh = plsc.VectorSubcoreMesh(core_axis_name="c", subcore_axis_name="s")
    ity = pltpu.VMEM((_CH,), jnp.int32)
    rty = pltpu.VMEM((_CH, HIDDEN), jnp.float32)
    k = pl.kernel(
        functools.partial(_sc_body, ntok),
        out_type=jax.ShapeDtypeStruct((ntok, HIDDEN), jnp.float32),
        mesh=mesh,
        scratch_types=[
            ity, ity, ity, ity, ity, ity,
            rty, rty, rty, rty, rty, rty,
            pltpu.SemaphoreType.DMA,
            pltpu.SemaphoreType.DMA,
            pltpu.SemaphoreType.DMA,
            pltpu.SemaphoreType.DMA,
            pltpu.SemaphoreType.DMA,
            pltpu.SemaphoreType.DMA,
        ],
    )
    return k(i0, i1, i2, t0, t1, t2)


def kernel(input, pos_emb, emb0, emb1, emb2, Wt, Wp, Wproj, bproj):
    b, s, f = input.shape
    ntok = b * s
    in2d = input.reshape(ntok, f)
    w0 = Wproj[:, 102:153].T
    w1 = Wproj[:, 153:204].T
    w2 = Wproj[:, 204:255].T
    t0, t1, t2 = _project_tables(emb0, emb1, emb2, w0, w1, w2)
    lanes = jnp.arange(128)
    cols = jnp.arange(640)
    sel = [(cols[:, None] == lanes[None, :] * 5 + c).astype(jnp.float32)
           for c in (1, 2, 3)]
    i0, i1, i2 = _split_input(input.reshape(ntok // 128, 128 * f),
                              sel[0], sel[1], sel[2], ntok)
    hi = lax.Precision.HIGHEST
    vt = jnp.dot(Wproj[:, 0:51], Wt, precision=hi).reshape(1, HIDDEN)
    vp = jnp.dot(Wproj[:, 51:102], Wp, precision=hi).reshape(1, HIDDEN)
    vtp = jnp.concatenate([vt, vp], axis=0)
    base = pos_emb[:s] + bproj[None, :]
    g = _sc_gather_sum(i0, i1, i2, t0, t1, t2, ntok)
    out = _epilogue(g, in2d, vtp, base, ntok)
    return out.reshape(b, s, HIDDEN)


# speedup vs baseline: 1.2156x; 1.2156x over previous
import functools

import jax
import jax.numpy as jnp
from jax import lax
from jax.experimental import pallas as pl
from jax.experimental.pallas import tpu as pltpu
from jax.experimental.pallas import tpu_sc as plsc

HIDDEN = 128
VOCAB = 65539
EMB = 51

_NC, _NS, _L = 2, 16, 16
_NW = _NC * _NS
_CH = 128
_G = HIDDEN // _L


def _proj_body(e0, e1, e2, w0, w1, w2, o0, o1, o2):
    dn = (((1,), (0,)), ((), ()))
    o0[...] = lax.dot_general(e0[...], w0[...], dn,
                              preferred_element_type=jnp.float32)
    o1[...] = lax.dot_general(e1[...], w1[...], dn,
                              preferred_element_type=jnp.float32)
    o2[...] = lax.dot_general(e2[...], w2[...], dn,
                              preferred_element_type=jnp.float32)


def _project_tables(emb0, emb1, emb2, w0, w1, w2):
    R = 4096
    nblk = (VOCAB + R - 1) // R
    espec = pl.BlockSpec((R, EMB), lambda i: (i, 0))
    wspec = pl.BlockSpec((EMB, HIDDEN), lambda i: (0, 0))
    ospec = pl.BlockSpec((R, HIDDEN), lambda i: (i, 0))
    oshape = jax.ShapeDtypeStruct((VOCAB, HIDDEN), jnp.float32)
    return pl.pallas_call(
        _proj_body,
        grid=(nblk,),
        in_specs=[espec, espec, espec, wspec, wspec, wspec],
        out_specs=[ospec, ospec, ospec],
        out_shape=[oshape, oshape, oshape],
    )(emb0, emb1, emb2, w0, w1, w2)


def _epilogue_body(g_ref, x_ref, vtp_ref, base_ref, o_ref):
    g = g_ref[...]
    x = x_ref[...]
    rt = g.shape[0]
    acc = g + x[:, 0:1] * vtp_ref[0:1, :] + x[:, 4:5] * vtp_ref[1:2, :]
    acc = (acc.reshape(rt // 32, 32, HIDDEN) + base_ref[...][None, :, :])
    o_ref[...] = acc.reshape(rt, HIDDEN)


def _epilogue(g, in2d, vtp, base, ntok):
    RT = 8192
    return pl.pallas_call(
        _epilogue_body,
        grid=(ntok // RT,),
        in_specs=[
            pl.BlockSpec((RT, HIDDEN), lambda i: (i, 0)),
            pl.BlockSpec((RT, 5), lambda i: (i, 0)),
            pl.BlockSpec((2, HIDDEN), lambda i: (0, 0)),
            pl.BlockSpec((32, HIDDEN), lambda i: (0, 0)),
        ],
        out_specs=pl.BlockSpec((RT, HIDDEN), lambda i: (i, 0)),
        out_shape=jax.ShapeDtypeStruct((ntok, HIDDEN), jnp.float32),
    )(g, in2d, vtp, base)


def _sc_body(ntok, i0_hbm, i1_hbm, i2_hbm, t0_hbm, t1_hbm, t2_hbm, out_hbm,
             ia0, ib0, ia1, ib1, ia2, ib2,
             ra0, rb0, ra1, rb1, ra2, rb2,
             gsa, gsb, osa, osb, isem):
    cid = lax.axis_index("c")
    sid = lax.axis_index("s")
    wid = sid * _NC + cid
    tpw = ntok // _NW
    nchunk = tpw // _CH
    tok0 = wid * tpw

    ibufs = ((ia0, ia1, ia2), (ib0, ib1, ib2))
    rbufs = ((ra0, ra1, ra2), (rb0, rb1, rb2))
    tabs = (t0_hbm, t1_hbm, t2_hbm)
    gsems = (gsa, gsb)
    osems = (osa, osb)

    def load_idx(j, bufs):
        sl = pl.ds(tok0 + j * _CH, _CH)
        d0 = pltpu.async_copy(i0_hbm.at[sl], bufs[0], isem)
        d1 = pltpu.async_copy(i1_hbm.at[sl], bufs[1], isem)
        d2 = pltpu.async_copy(i2_hbm.at[sl], bufs[2], isem)
        d0.wait()
        d1.wait()
        d2.wait()

    def start_gathers(p):
        for k in range(3):
            pltpu.async_copy(tabs[k].at[ibufs[p][k]], rbufs[p][k], gsems[p])

    def wait_gathers(p):
        for k in range(3):
            pltpu.make_async_copy(tabs[k].at[ibufs[p][k]], rbufs[p][k],
                                  gsems[p]).wait()

    def start_out(j, p):
        pltpu.async_copy(rbufs[p][0], out_hbm.at[pl.ds(tok0 + j * _CH, _CH)],
                         osems[p])

    def wait_out(j, p):
        pltpu.make_async_copy(rbufs[p][0],
                              out_hbm.at[pl.ds(tok0 + j * _CH, _CH)],
                              osems[p]).wait()

    load_idx(0, ibufs[0])
    start_gathers(0)

    npair = nchunk // 2

    def pair_body(jp, carry):
        for p in range(2):
            j = jp * 2 + p
            q = 1 - p

            @pl.when(j + 1 < nchunk)
            def _():
                load_idx(j + 1, ibufs[q])

                @pl.when(j >= 1)
                def _():
                    wait_out(j - 1, q)
                start_gathers(q)

            wait_gathers(p)

            def tok_body(r, carry2):
                for g in range(_G):
                    ds = pl.ds(g * _L, _L)
                    rbufs[p][0][r, ds] = (rbufs[p][0][r, ds]
                                          + rbufs[p][1][r, ds]
                                          + rbufs[p][2][r, ds])
                return carry2

            lax.fori_loop(0, _CH, tok_body, 0)
            start_out(j, p)
        return carry

    lax.fori_loop(0, npair, pair_body, 0)
    wait_out(nchunk - 2, 0)
    wait_out(nchunk - 1, 1)


def _sc_gather_sum(i0, i1, i2, t0, t1, t2, ntok):
    mesh = plsc.VectorSubcoreMesh(core_axis_name="c", subcore_axis_name="s")
    ity = pltpu.VMEM((_CH,), jnp.int32)
    rty = pltpu.VMEM((_CH, HIDDEN), jnp.float32)
    k = pl.kernel(
        functools.partial(_sc_body, ntok),
        out_type=jax.ShapeDtypeStruct((ntok, HIDDEN), jnp.float32),
        mesh=mesh,
        scratch_types=[
            ity, ity, ity, ity, ity, ity,
            rty, rty, rty, rty, rty, rty,
            pltpu.SemaphoreType.DMA,
            pltpu.SemaphoreType.DMA,
            pltpu.SemaphoreType.DMA,
            pltpu.SemaphoreType.DMA,
            pltpu.SemaphoreType.DMA,
        ],
    )
    return k(i0, i1, i2, t0, t1, t2)


def kernel(input, pos_emb, emb0, emb1, emb2, Wt, Wp, Wproj, bproj):
    b, s, f = input.shape
    ntok = b * s
    in2d = input.reshape(ntok, f)
    w0 = Wproj[:, 102:153].T
    w1 = Wproj[:, 153:204].T
    w2 = Wproj[:, 204:255].T
    t0, t1, t2 = _project_tables(emb0, emb1, emb2, w0, w1, w2)
    i0 = input[:, :, 1].astype(jnp.int32).reshape(ntok)
    i1 = input[:, :, 2].astype(jnp.int32).reshape(ntok)
    i2 = input[:, :, 3].astype(jnp.int32).reshape(ntok)
    hi = lax.Precision.HIGHEST
    vt = jnp.dot(Wproj[:, 0:51], Wt, precision=hi).reshape(1, HIDDEN)
    vp = jnp.dot(Wproj[:, 51:102], Wp, precision=hi).reshape(1, HIDDEN)
    vtp = jnp.concatenate([vt, vp], axis=0)
    base = pos_emb[:s] + bproj[None, :]
    g = _sc_gather_sum(i0, i1, i2, t0, t1, t2, ntok)
    out = _epilogue(g, in2d, vtp, base, ntok)
    return out.reshape(b, s, HIDDEN)


# speedup vs baseline: 1.2366x; 1.0173x over previous
import functools

import jax
import jax.numpy as jnp
from jax import lax
from jax.experimental import pallas as pl
from jax.experimental.pallas import tpu as pltpu
from jax.experimental.pallas import tpu_sc as plsc

HIDDEN = 128
VOCAB = 65539
EMB = 51

_NC, _NS, _L = 2, 16, 16
_NW = _NC * _NS
_CH = 128
_G = HIDDEN // _L


def _proj_body(e0, e1, e2, w0, w1, w2, o0, o1, o2):
    dn = (((1,), (0,)), ((), ()))
    o0[...] = lax.dot_general(e0[...], w0[...], dn,
                              preferred_element_type=jnp.float32)
    o1[...] = lax.dot_general(e1[...], w1[...], dn,
                              preferred_element_type=jnp.float32)
    o2[...] = lax.dot_general(e2[...], w2[...], dn,
                              preferred_element_type=jnp.float32)


def _project_tables(emb0, emb1, emb2, w0, w1, w2):
    R = 8192
    nblk = (VOCAB + R - 1) // R
    espec = pl.BlockSpec((R, EMB), lambda i: (i, 0))
    wspec = pl.BlockSpec((EMB, HIDDEN), lambda i: (0, 0))
    ospec = pl.BlockSpec((R, HIDDEN), lambda i: (i, 0))
    oshape = jax.ShapeDtypeStruct((VOCAB, HIDDEN), jnp.float32)
    return pl.pallas_call(
        _proj_body,
        grid=(nblk,),
        in_specs=[espec, espec, espec, wspec, wspec, wspec],
        out_specs=[ospec, ospec, ospec],
        out_shape=[oshape, oshape, oshape],
    )(emb0, emb1, emb2, w0, w1, w2)


def _epilogue_body(g_ref, x_ref, vtp_ref, base_ref, o_ref):
    g = g_ref[...]
    x = x_ref[...]
    rt = g.shape[0]
    acc = g + x[:, 0:1] * vtp_ref[0:1, :] + x[:, 4:5] * vtp_ref[1:2, :]
    acc = (acc.reshape(rt // 32, 32, HIDDEN) + base_ref[...][None, :, :])
    o_ref[...] = acc.reshape(rt, HIDDEN)


def _epilogue(g, in2d, vtp, base, ntok):
    RT = 16384
    return pl.pallas_call(
        _epilogue_body,
        grid=(ntok // RT,),
        in_specs=[
            pl.BlockSpec((RT, HIDDEN), lambda i: (i, 0)),
            pl.BlockSpec((RT, 5), lambda i: (i, 0)),
            pl.BlockSpec((2, HIDDEN), lambda i: (0, 0)),
            pl.BlockSpec((32, HIDDEN), lambda i: (0, 0)),
        ],
        out_specs=pl.BlockSpec((RT, HIDDEN), lambda i: (i, 0)),
        out_shape=jax.ShapeDtypeStruct((ntok, HIDDEN), jnp.float32),
    )(g, in2d, vtp, base)


def _sc_body(ntok, i0_hbm, i1_hbm, i2_hbm, t0_hbm, t1_hbm, t2_hbm, out_hbm,
             ia0, ib0, ia1, ib1, ia2, ib2,
             ra0, rb0, ra1, rb1, ra2, rb2,
             gsa, gsb, osa, osb, isem):
    cid = lax.axis_index("c")
    sid = lax.axis_index("s")
    wid = sid * _NC + cid
    tpw = ntok // _NW
    nchunk = tpw // _CH
    tok0 = wid * tpw

    ibufs = ((ia0, ia1, ia2), (ib0, ib1, ib2))
    rbufs = ((ra0, ra1, ra2), (rb0, rb1, rb2))
    tabs = (t0_hbm, t1_hbm, t2_hbm)
    gsems = (gsa, gsb)
    osems = (osa, osb)

    def load_idx(j, bufs):
        sl = pl.ds(tok0 + j * _CH, _CH)
        d0 = pltpu.async_copy(i0_hbm.at[sl], bufs[0], isem)
        d1 = pltpu.async_copy(i1_hbm.at[sl], bufs[1], isem)
        d2 = pltpu.async_copy(i2_hbm.at[sl], bufs[2], isem)
        d0.wait()
        d1.wait()
        d2.wait()

    def start_gathers(p):
        for k in range(3):
            pltpu.async_copy(tabs[k].at[ibufs[p][k]], rbufs[p][k], gsems[p])

    def wait_gathers(p):
        for k in range(3):
            pltpu.make_async_copy(tabs[k].at[ibufs[p][k]], rbufs[p][k],
                                  gsems[p]).wait()

    def start_out(j, p):
        pltpu.async_copy(rbufs[p][0], out_hbm.at[pl.ds(tok0 + j * _CH, _CH)],
                         osems[p])

    def wait_out(j, p):
        pltpu.make_async_copy(rbufs[p][0],
                              out_hbm.at[pl.ds(tok0 + j * _CH, _CH)],
                              osems[p]).wait()

    load_idx(0, ibufs[0])
    start_gathers(0)

    npair = nchunk // 2

    def pair_body(jp, carry):
        for p in range(2):
            j = jp * 2 + p
            q = 1 - p

            @pl.when(j + 1 < nchunk)
            def _():
                load_idx(j + 1, ibufs[q])

                @pl.when(j >= 1)
                def _():
                    wait_out(j - 1, q)
                start_gathers(q)

            wait_gathers(p)

            def tok_body(r, carry2):
                for g in range(_G):
                    ds = pl.ds(g * _L, _L)
                    rbufs[p][0][r, ds] = (rbufs[p][0][r, ds]
                                          + rbufs[p][1][r, ds]
                                          + rbufs[p][2][r, ds])
                return carry2

            lax.fori_loop(0, _CH, tok_body, 0)
            start_out(j, p)
        return carry

    lax.fori_loop(0, npair, pair_body, 0)
    wait_out(nchunk - 2, 0)
    wait_out(nchunk - 1, 1)


def _sc_gather_sum(i0, i1, i2, t0, t1, t2, ntok):
    mesh = plsc.VectorSubcoreMesh(core_axis_name="c", subcore_axis_name="s")
    ity = pltpu.VMEM((_CH,), jnp.int32)
    rty = pltpu.VMEM((_CH, HIDDEN), jnp.float32)
    k = pl.kernel(
        functools.partial(_sc_body, ntok),
        out_type=jax.ShapeDtypeStruct((ntok, HIDDEN), jnp.float32),
        mesh=mesh,
        scratch_types=[
            ity, ity, ity, ity, ity, ity,
            rty, rty, rty, rty, rty, rty,
            pltpu.SemaphoreType.DMA,
            pltpu.SemaphoreType.DMA,
            pltpu.SemaphoreType.DMA,
            pltpu.SemaphoreType.DMA,
            pltpu.SemaphoreType.DMA,
        ],
    )
    return k(i0, i1, i2, t0, t1, t2)


def kernel(input, pos_emb, emb0, emb1, emb2, Wt, Wp, Wproj, bproj):
    b, s, f = input.shape
    ntok = b * s
    in2d = input.reshape(ntok, f)
    w0 = Wproj[:, 102:153].T
    w1 = Wproj[:, 153:204].T
    w2 = Wproj[:, 204:255].T
    t0, t1, t2 = _project_tables(emb0, emb1, emb2, w0, w1, w2)
    i0 = input[:, :, 1].astype(jnp.int32).reshape(ntok)
    i1 = input[:, :, 2].astype(jnp.int32).reshape(ntok)
    i2 = input[:, :, 3].astype(jnp.int32).reshape(ntok)
    hi = lax.Precision.HIGHEST
    vt = jnp.dot(Wproj[:, 0:51], Wt, precision=hi).reshape(1, HIDDEN)
    vp = jnp.dot(Wproj[:, 51:102], Wp, precision=hi).reshape(1, HIDDEN)
    vtp = jnp.concatenate([vt, vp], axis=0)
    base = pos_emb[:s] + bproj[None, :]
    g = _sc_gather_sum(i0, i1, i2, t0, t1, t2, ntok)
    out = _epilogue(g, in2d, vtp, base, ntok)
    return out.reshape(b, s, HIDDEN)
